# Initial kernel scaffold; baseline (speedup 1.0000x reference)
import jax
import jax.numpy as jnp
from jax.experimental import pallas as pl


def kernel(image, boxes, box_ind):
    raise NotImplementedError("write your pallas kernel here")


# speedup vs baseline: 1.1125x; 1.1125x over previous
import functools

import jax
import jax.numpy as jnp
from jax import lax
from jax.experimental import pallas as pl
from jax.experimental.pallas import tpu as pltpu
from jax.experimental.pallas import tpu_sc as plsc

B, C, H, W = 4, 96, 224, 224
CROP_H, CROP_W = 14, 14
N_BOX = 1024
NPIX = N_BOX * CROP_H * CROP_W
HW = H * W

NC, NS, LANES = 2, 16, 16
NW = NC * NS
ROWS_PER_W = NPIX // NW
CHUNK = 128
NCHUNK = ROWS_PER_W // CHUNK
NVREG = C // LANES
TABLE_D = 128


def _index_body(boxes_ref, bind_ref, ptl_ref, ptr_ref, pbl_ref, pbr_ref,
                wtl_ref, wtr_ref, wbl_ref, wbr_ref):
    f32 = jnp.float32
    boxes = boxes_ref[...]
    bind = bind_ref[...]
    y1 = boxes[:, 0:1]
    x1 = boxes[:, 1:2]
    y2 = boxes[:, 2:3]
    x2 = boxes[:, 3:4]
    col = lax.broadcasted_iota(jnp.int32, (N_BOX, CROP_H * CROP_W), 1)
    i_f = (col // CROP_W).astype(f32)
    j_f = (col % CROP_W).astype(f32)
    height_scale = (y2 - y1) * (H - 1) / (CROP_H - 1)
    width_scale = (x2 - x1) * (W - 1) / (CROP_W - 1)
    in_y = y1 * (H - 1) + i_f * height_scale
    in_x = x1 * (W - 1) + j_f * width_scale
    ti = in_y.astype(jnp.int32)
    li = in_x.astype(jnp.int32)
    yl = in_y - ti.astype(f32)
    xl = in_x - li.astype(f32)
    p_tl = bind * HW + ti * W + li
    ptl_ref[...] = p_tl
    ptr_ref[...] = p_tl + 1
    pbl_ref[...] = p_tl + W
    pbr_ref[...] = p_tl + (W + 1)
    one = f32(1.0)
    wtl_ref[...] = (one - yl) * (one - xl)
    wtr_ref[...] = (one - yl) * xl
    wbl_ref[...] = yl * (one - xl)
    wbr_ref[...] = yl * xl


def _compute_indices(boxes, box_ind):
    shp = jax.ShapeDtypeStruct((N_BOX, CROP_H * CROP_W), jnp.int32)
    shpf = jax.ShapeDtypeStruct((N_BOX, CROP_H * CROP_W), jnp.float32)
    return pl.pallas_call(
        _index_body,
        out_shape=(shp, shp, shp, shp, shpf, shpf, shpf, shpf),
    )(boxes, box_ind.reshape(N_BOX, 1))


def _sc_body(table, ptl, ptr, pbl, pbr, wtl, wtr, wbl, wbr, out,
             itl, itr, ibl, ibr, vtl, vtr, vbl, vbr,
             rtl, rtr, rbl, rbr, rout, sem):
    wid = lax.axis_index("s") * NC + lax.axis_index("c")
    base0 = pl.multiple_of(wid * ROWS_PER_W, CHUNK)

    def chunk_body(ci, carry):
        base = pl.multiple_of(base0 + ci * CHUNK, CHUNK)
        rows = pl.ds(base, CHUNK)
        pltpu.sync_copy(ptl.at[rows], itl)
        pltpu.sync_copy(ptr.at[rows], itr)
        pltpu.sync_copy(pbl.at[rows], ibl)
        pltpu.sync_copy(pbr.at[rows], ibr)
        pltpu.sync_copy(wtl.at[rows], vtl)
        pltpu.sync_copy(wtr.at[rows], vtr)
        pltpu.sync_copy(wbl.at[rows], vbl)
        pltpu.sync_copy(wbr.at[rows], vbr)
        c1 = pltpu.async_copy(table.at[itl], rtl, sem)
        c2 = pltpu.async_copy(table.at[itr], rtr, sem)
        c3 = pltpu.async_copy(table.at[ibl], rbl, sem)
        c4 = pltpu.async_copy(table.at[ibr], rbr, sem)
        c1.wait()
        c2.wait()
        c3.wait()
        c4.wait()

        def group_body(g, gcarry):
            r0 = pl.multiple_of(g * LANES, LANES)
            wa = vtl[pl.ds(r0, LANES)]
            wb = vtr[pl.ds(r0, LANES)]
            wc = vbl[pl.ds(r0, LANES)]
            wd = vbr[pl.ds(r0, LANES)]
            for k in range(LANES):
                r = r0 + k
                a, b, c, d = wa[k], wb[k], wc[k], wd[k]
                for v in range(NVREG):
                    sl = pl.ds(v * LANES, LANES)
                    rout[r, sl] = (rtl[r, sl] * a + rtr[r, sl] * b
                                   + rbl[r, sl] * c + rbr[r, sl] * d)
            return gcarry

        lax.fori_loop(0, CHUNK // LANES, group_body, 0, unroll=False)
        pltpu.sync_copy(rout, out.at[rows])
        return carry

    lax.fori_loop(0, NCHUNK, chunk_body, 0, unroll=False)


@functools.cache
def _make_sc_gather():
    return functools.partial(
        pl.kernel,
        out_type=jax.ShapeDtypeStruct((NPIX, C), jnp.float32),
        mesh=plsc.VectorSubcoreMesh(
            core_axis_name="c", subcore_axis_name="s",
            num_cores=NC, num_subcores=NS),
        scratch_types=(
            [pltpu.VMEM((CHUNK,), jnp.int32)] * 4
            + [pltpu.VMEM((CHUNK,), jnp.float32)] * 4
            + [pltpu.VMEM((CHUNK, TABLE_D), jnp.float32)] * 4
            + [pltpu.VMEM((CHUNK, C), jnp.float32)]
            + [pltpu.SemaphoreType.DMA]
        ),
    )(_sc_body)


def kernel(image, boxes, box_ind):
    table = jnp.pad(jnp.transpose(image, (0, 2, 3, 1)).reshape(B * HW, C),
                    ((0, 0), (0, TABLE_D - C)))
    ptl, ptr, pbl, pbr, wtl, wtr, wbl, wbr = _compute_indices(boxes, box_ind)
    flat = lambda a: a.reshape(NPIX)
    out_t = _make_sc_gather()(table, flat(ptl), flat(ptr), flat(pbl),
                              flat(pbr), flat(wtl), flat(wtr), flat(wbl),
                              flat(wbr))
    return jnp.transpose(out_t.reshape(N_BOX, CROP_H, CROP_W, C),
                         (0, 3, 1, 2))


# speedup vs baseline: 1.4888x; 1.3383x over previous
import functools

import jax
import jax.numpy as jnp
from jax import lax
from jax.experimental import pallas as pl
from jax.experimental.pallas import tpu as pltpu
from jax.experimental.pallas import tpu_sc as plsc

B, C, H, W = 4, 96, 224, 224
CROP_H, CROP_W = 14, 14
N_BOX = 1024
NPIX = N_BOX * CROP_H * CROP_W
HW = H * W

NC, NS, LANES = 2, 16, 16
NW = NC * NS
ROWS_PER_W = NPIX // NW
CHUNK = 64
NCHUNK = ROWS_PER_W // CHUNK
NVREG = C // LANES
TABLE_D = 128


def _index_body(boxes_ref, bind_ref, ptl_ref, ptr_ref, pbl_ref, pbr_ref,
                wtl_ref, wtr_ref, wbl_ref, wbr_ref):
    f32 = jnp.float32
    boxes = boxes_ref[...]
    bind = bind_ref[...]
    y1 = boxes[:, 0:1]
    x1 = boxes[:, 1:2]
    y2 = boxes[:, 2:3]
    x2 = boxes[:, 3:4]
    col = lax.broadcasted_iota(jnp.int32, (N_BOX, CROP_H * CROP_W), 1)
    i_f = (col // CROP_W).astype(f32)
    j_f = (col % CROP_W).astype(f32)
    height_scale = (y2 - y1) * (H - 1) / (CROP_H - 1)
    width_scale = (x2 - x1) * (W - 1) / (CROP_W - 1)
    in_y = y1 * (H - 1) + i_f * height_scale
    in_x = x1 * (W - 1) + j_f * width_scale
    ti = in_y.astype(jnp.int32)
    li = in_x.astype(jnp.int32)
    yl = in_y - ti.astype(f32)
    xl = in_x - li.astype(f32)
    p_tl = bind * HW + ti * W + li
    ptl_ref[...] = p_tl
    ptr_ref[...] = p_tl + 1
    pbl_ref[...] = p_tl + W
    pbr_ref[...] = p_tl + (W + 1)
    one = f32(1.0)
    wtl_ref[...] = (one - yl) * (one - xl)
    wtr_ref[...] = (one - yl) * xl
    wbl_ref[...] = yl * (one - xl)
    wbr_ref[...] = yl * xl


def _compute_indices(boxes, box_ind):
    shp = jax.ShapeDtypeStruct((N_BOX, CROP_H * CROP_W), jnp.int32)
    shpf = jax.ShapeDtypeStruct((N_BOX, CROP_H * CROP_W), jnp.float32)
    return pl.pallas_call(
        _index_body,
        out_shape=(shp, shp, shp, shp, shpf, shpf, shpf, shpf),
    )(boxes, box_ind.reshape(N_BOX, 1))


def _sc_body(table, ptl, ptr, pbl, pbr, wtl, wtr, wbl, wbr, out,
             itl, itr, ibl, ibr, vtl, vtr, vbl, vbr,
             bufs0, bufs1, rout, sem0, sem1):
    wid = lax.axis_index("s") * NC + lax.axis_index("c")
    base0 = pl.multiple_of(wid * ROWS_PER_W, CHUNK)
    span = pl.ds(base0, ROWS_PER_W)

    pltpu.sync_copy(ptl.at[span], itl)
    pltpu.sync_copy(ptr.at[span], itr)
    pltpu.sync_copy(pbl.at[span], ibl)
    pltpu.sync_copy(pbr.at[span], ibr)
    pltpu.sync_copy(wtl.at[span], vtl)
    pltpu.sync_copy(wtr.at[span], vtr)
    pltpu.sync_copy(wbl.at[span], vbl)
    pltpu.sync_copy(wbr.at[span], vbr)

    idx_refs = (itl, itr, ibl, ibr)
    w_refs = (vtl, vtr, vbl, vbr)
    ring = ((bufs0, sem0), (bufs1, sem1))

    def fire(ci, bufs, sem):
        off = pl.multiple_of(ci * CHUNK, CHUNK)
        for k in range(4):
            pltpu.async_copy(table.at[idx_refs[k].at[pl.ds(off, CHUNK)]],
                             bufs.at[k], sem)

    def drain(bufs, sem):
        for k in range(4):
            pltpu.make_async_copy(table.at[pl.ds(0, CHUNK)],
                                  bufs.at[k], sem).wait()

    def blend(ci, bufs):
        off = pl.multiple_of(ci * CHUNK, CHUNK)

        def group_body(g, gcarry):
            r0 = pl.multiple_of(g * LANES, LANES)
            wa = vtl[pl.ds(off + r0, LANES)]
            wb = vtr[pl.ds(off + r0, LANES)]
            wc = vbl[pl.ds(off + r0, LANES)]
            wd = vbr[pl.ds(off + r0, LANES)]
            for k in range(LANES):
                r = r0 + k
                a, b, c, d = wa[k], wb[k], wc[k], wd[k]
                for v in range(NVREG):
                    sl = pl.ds(v * LANES, LANES)
                    rout[r, sl] = (bufs[0, r, sl] * a + bufs[1, r, sl] * b
                                   + bufs[2, r, sl] * c + bufs[3, r, sl] * d)
            return gcarry

        lax.fori_loop(0, CHUNK // LANES, group_body, 0, unroll=False)
        pltpu.sync_copy(rout, out.at[pl.ds(base0 + off, CHUNK)])

    fire(0, *ring[0])

    def pair_body(g, carry):
        c0 = pl.multiple_of(g * 2, 2)
        fire(c0 + 1, *ring[1])
        drain(*ring[0])
        blend(c0, ring[0][0])

        @pl.when(c0 + 2 < NCHUNK)
        def _():
            fire(c0 + 2, *ring[0])

        drain(*ring[1])
        blend(c0 + 1, ring[1][0])
        return carry

    lax.fori_loop(0, NCHUNK // 2, pair_body, 0, unroll=False)


@functools.cache
def _make_sc_gather():
    return functools.partial(
        pl.kernel,
        out_type=jax.ShapeDtypeStruct((NPIX, C), jnp.float32),
        mesh=plsc.VectorSubcoreMesh(
            core_axis_name="c", subcore_axis_name="s",
            num_cores=NC, num_subcores=NS),
        scratch_types=(
            [pltpu.VMEM((ROWS_PER_W,), jnp.int32)] * 4
            + [pltpu.VMEM((ROWS_PER_W,), jnp.float32)] * 4
            + [pltpu.VMEM((4, CHUNK, TABLE_D), jnp.float32)] * 2
            + [pltpu.VMEM((CHUNK, C), jnp.float32)]
            + [pltpu.SemaphoreType.DMA] * 2
        ),
    )(_sc_body)


def kernel(image, boxes, box_ind):
    table = jnp.pad(jnp.transpose(image, (0, 2, 3, 1)).reshape(B * HW, C),
                    ((0, 0), (0, TABLE_D - C)))
    ptl, ptr, pbl, pbr, wtl, wtr, wbl, wbr = _compute_indices(boxes, box_ind)
    flat = lambda a: a.reshape(NPIX)
    out_t = _make_sc_gather()(table, flat(ptl), flat(ptr), flat(pbl),
                              flat(pbr), flat(wtl), flat(wtr), flat(wbl),
                              flat(wbr))
    return jnp.transpose(out_t.reshape(N_BOX, CROP_H, CROP_W, C),
                         (0, 3, 1, 2))


# speedup vs baseline: 2.1524x; 1.4458x over previous
import functools

import jax
import jax.numpy as jnp
from jax import lax
from jax.experimental import pallas as pl
from jax.experimental.pallas import tpu as pltpu
from jax.experimental.pallas import tpu_sc as plsc

B, C, H, W = 4, 96, 224, 224
CROP_H, CROP_W = 14, 14
N_BOX = 1024
NPIX = N_BOX * CROP_H * CROP_W
HW = H * W

NC, NS, LANES = 2, 16, 16
NW = NC * NS
ROWS_PER_W = NPIX // NW
CHUNK = 64
NCHUNK = ROWS_PER_W // CHUNK
NVREG = C // LANES
TABLE_D = 128


def _index_body(boxes_ref, bind_ref, ptl_ref, ptr_ref, pbl_ref, pbr_ref,
                wtl_ref, wtr_ref, wbl_ref, wbr_ref):
    f32 = jnp.float32
    boxes = boxes_ref[...]
    bind = bind_ref[...]
    y1 = boxes[:, 0:1]
    x1 = boxes[:, 1:2]
    y2 = boxes[:, 2:3]
    x2 = boxes[:, 3:4]
    col = lax.broadcasted_iota(jnp.int32, (N_BOX, CROP_H * CROP_W), 1)
    i_f = (col // CROP_W).astype(f32)
    j_f = (col % CROP_W).astype(f32)
    height_scale = (y2 - y1) * (H - 1) / (CROP_H - 1)
    width_scale = (x2 - x1) * (W - 1) / (CROP_W - 1)
    in_y = y1 * (H - 1) + i_f * height_scale
    in_x = x1 * (W - 1) + j_f * width_scale
    ti = in_y.astype(jnp.int32)
    li = in_x.astype(jnp.int32)
    yl = in_y - ti.astype(f32)
    xl = in_x - li.astype(f32)
    p_tl = bind * HW + ti * W + li
    ptl_ref[...] = p_tl
    ptr_ref[...] = p_tl + 1
    pbl_ref[...] = p_tl + W
    pbr_ref[...] = p_tl + (W + 1)
    one = f32(1.0)
    wtl_ref[...] = (one - yl) * (one - xl)
    wtr_ref[...] = (one - yl) * xl
    wbl_ref[...] = yl * (one - xl)
    wbr_ref[...] = yl * xl


YB = 8
BB = 32


def _in_transpose_body(img_ref, tab_ref):
    x = img_ref[0].reshape(C, YB * W)
    xt = jnp.transpose(x, (1, 0))
    pad = jnp.zeros((YB * W, TABLE_D - C), jnp.float32)
    tab_ref[...] = jnp.concatenate([xt, pad], axis=1)


def _make_table(image):
    return pl.pallas_call(
        _in_transpose_body,
        grid=(B, H // YB),
        in_specs=[pl.BlockSpec((1, C, YB, W), lambda b, y: (b, 0, y, 0))],
        out_specs=pl.BlockSpec((YB * W, TABLE_D),
                               lambda b, y: (b * (H // YB) + y, 0)),
        out_shape=jax.ShapeDtypeStruct((B * HW, TABLE_D), jnp.float32),
    )(image)


def _out_transpose_body(rows_ref, out_ref):
    x = rows_ref[...].reshape(BB, CROP_H * CROP_W, C)
    out_ref[...] = jnp.transpose(x, (0, 2, 1))


def _to_nchw(out_t):
    npb = CROP_H * CROP_W
    return pl.pallas_call(
        _out_transpose_body,
        grid=(N_BOX // BB,),
        in_specs=[pl.BlockSpec((BB * npb, C), lambda i: (i, 0))],
        out_specs=pl.BlockSpec((BB, C, npb), lambda i: (i, 0, 0)),
        out_shape=jax.ShapeDtypeStruct((N_BOX, C, npb), jnp.float32),
    )(out_t)


def _compute_indices(boxes, box_ind):
    shp = jax.ShapeDtypeStruct((N_BOX, CROP_H * CROP_W), jnp.int32)
    shpf = jax.ShapeDtypeStruct((N_BOX, CROP_H * CROP_W), jnp.float32)
    return pl.pallas_call(
        _index_body,
        out_shape=(shp, shp, shp, shp, shpf, shpf, shpf, shpf),
    )(boxes, box_ind.reshape(N_BOX, 1))


def _sc_body(table, ptl, ptr, pbl, pbr, wtl, wtr, wbl, wbr, out,
             itl, itr, ibl, ibr, vtl, vtr, vbl, vbr,
             bufs0, bufs1, rout, sem0, sem1):
    wid = lax.axis_index("s") * NC + lax.axis_index("c")
    base0 = pl.multiple_of(wid * ROWS_PER_W, CHUNK)
    span = pl.ds(base0, ROWS_PER_W)

    pltpu.sync_copy(ptl.at[span], itl)
    pltpu.sync_copy(ptr.at[span], itr)
    pltpu.sync_copy(pbl.at[span], ibl)
    pltpu.sync_copy(pbr.at[span], ibr)
    pltpu.sync_copy(wtl.at[span], vtl)
    pltpu.sync_copy(wtr.at[span], vtr)
    pltpu.sync_copy(wbl.at[span], vbl)
    pltpu.sync_copy(wbr.at[span], vbr)

    idx_refs = (itl, itr, ibl, ibr)
    w_refs = (vtl, vtr, vbl, vbr)
    ring = ((bufs0, sem0), (bufs1, sem1))

    def fire(ci, bufs, sem):
        off = pl.multiple_of(ci * CHUNK, CHUNK)
        for k in range(4):
            pltpu.async_copy(table.at[idx_refs[k].at[pl.ds(off, CHUNK)]],
                             bufs.at[k], sem)

    def drain(bufs, sem):
        for k in range(4):
            pltpu.make_async_copy(table.at[pl.ds(0, CHUNK)],
                                  bufs.at[k], sem).wait()

    def blend(ci, bufs):
        off = pl.multiple_of(ci * CHUNK, CHUNK)

        def group_body(g, gcarry):
            r0 = pl.multiple_of(g * LANES, LANES)
            wa = vtl[pl.ds(off + r0, LANES)]
            wb = vtr[pl.ds(off + r0, LANES)]
            wc = vbl[pl.ds(off + r0, LANES)]
            wd = vbr[pl.ds(off + r0, LANES)]
            for k in range(LANES):
                r = r0 + k
                a, b, c, d = wa[k], wb[k], wc[k], wd[k]
                for v in range(NVREG):
                    sl = pl.ds(v * LANES, LANES)
                    rout[r, sl] = (bufs[0, r, sl] * a + bufs[1, r, sl] * b
                                   + bufs[2, r, sl] * c + bufs[3, r, sl] * d)
            return gcarry

        lax.fori_loop(0, CHUNK // LANES, group_body, 0, unroll=False)
        pltpu.sync_copy(rout, out.at[pl.ds(base0 + off, CHUNK)])

    fire(0, *ring[0])

    def pair_body(g, carry):
        c0 = pl.multiple_of(g * 2, 2)
        fire(c0 + 1, *ring[1])
        drain(*ring[0])
        blend(c0, ring[0][0])

        @pl.when(c0 + 2 < NCHUNK)
        def _():
            fire(c0 + 2, *ring[0])

        drain(*ring[1])
        blend(c0 + 1, ring[1][0])
        return carry

    lax.fori_loop(0, NCHUNK // 2, pair_body, 0, unroll=False)


@functools.cache
def _make_sc_gather():
    return functools.partial(
        pl.kernel,
        out_type=jax.ShapeDtypeStruct((NPIX, C), jnp.float32),
        mesh=plsc.VectorSubcoreMesh(
            core_axis_name="c", subcore_axis_name="s",
            num_cores=NC, num_subcores=NS),
        scratch_types=(
            [pltpu.VMEM((ROWS_PER_W,), jnp.int32)] * 4
            + [pltpu.VMEM((ROWS_PER_W,), jnp.float32)] * 4
            + [pltpu.VMEM((4, CHUNK, TABLE_D), jnp.float32)] * 2
            + [pltpu.VMEM((CHUNK, C), jnp.float32)]
            + [pltpu.SemaphoreType.DMA] * 2
        ),
    )(_sc_body)


def kernel(image, boxes, box_ind):
    table = _make_table(image)
    ptl, ptr, pbl, pbr, wtl, wtr, wbl, wbr = _compute_indices(boxes, box_ind)
    flat = lambda a: a.reshape(NPIX)
    out_t = _make_sc_gather()(table, flat(ptl), flat(ptr), flat(pbl),
                              flat(pbr), flat(wtl), flat(wtr), flat(wbl),
                              flat(wbr))
    return _to_nchw(out_t).reshape(N_BOX, C, CROP_H, CROP_W)
